# Initial kernel scaffold; baseline (speedup 1.0000x reference)
import jax
import jax.numpy as jnp
from jax.experimental import pallas as pl


def kernel(x, edge_index, W1, b1, W2, b2):
    raise NotImplementedError("write your pallas kernel here")


# speedup vs baseline: 11.1803x; 11.1803x over previous
import jax
import jax.numpy as jnp
from jax import lax
from jax.experimental import pallas as pl
from jax.experimental.pallas import tpu as pltpu
from jax.experimental.pallas import tpu_sc as plsc

N_NODES = 10000
D = 128
DH = 64
N_EDGES = 320000

NC = 2
NS = 16
NPAD = 10240
ROWS_PT = NPAD // NS
EPAD = 327680
ET = EPAD // NS
K = 128
NCHUNK = ET // K
NCHUNK_C = NCHUNK // NC
NBUF = 4

_mesh = plsc.VectorSubcoreMesh(core_axis_name="c", subcore_axis_name="s")


def _deg_body(src_hbm, dst_hbm, out_hbm, src_v, dst_v, ones_v, zrow_v,
              dsrc_sh, ddst_sh):
    cid = lax.axis_index("c")
    sid = lax.axis_index("s")

    o16 = jnp.ones((16,), jnp.float32)
    z16 = jnp.zeros((16,), jnp.float32)
    for l in range(K // 16):
        ones_v[pl.ds(l * 16, 16)] = o16
    for l in range(ROWS_PT // 16):
        zrow_v[pl.ds(l * 16, 16)] = z16

    pltpu.sync_copy(zrow_v, dsrc_sh.at[pl.ds(sid * ROWS_PT, ROWS_PT)])
    pltpu.sync_copy(zrow_v, ddst_sh.at[pl.ds(sid * ROWS_PT, ROWS_PT)])
    plsc.subcore_barrier()

    pltpu.sync_copy(src_hbm.at[sid].at[pl.ds(cid * NCHUNK_C, NCHUNK_C)], src_v)
    pltpu.sync_copy(dst_hbm.at[sid].at[pl.ds(cid * NCHUNK_C, NCHUNK_C)], dst_v)

    def chunk(j, _):
        pltpu.sync_copy(ones_v, dsrc_sh.at[src_v.at[j]], add=True)
        pltpu.sync_copy(ones_v, ddst_sh.at[dst_v.at[j]], add=True)
        return ()

    lax.fori_loop(0, NCHUNK_C, chunk, ())
    plsc.subcore_barrier()

    sl = pl.ds(sid * ROWS_PT, ROWS_PT)
    pltpu.sync_copy(dsrc_sh.at[sl], out_hbm.at[cid, 0, sl])
    pltpu.sync_copy(ddst_sh.at[sl], out_hbm.at[cid, 1, sl])


_deg_kernel = pl.kernel(
    _deg_body,
    out_type=jax.ShapeDtypeStruct((NC, 2, NPAD), jnp.float32),
    mesh=_mesh,
    scratch_types=[
        pltpu.VMEM((NCHUNK_C, K), jnp.int32),
        pltpu.VMEM((NCHUNK_C, K), jnp.int32),
        pltpu.VMEM((K,), jnp.float32),
        pltpu.VMEM((ROWS_PT,), jnp.float32),
        pltpu.VMEM_SHARED((NPAD,), jnp.float32),
        pltpu.VMEM_SHARED((NPAD,), jnp.float32),
    ],
)


def _spmm_body(z_hbm, src_hbm, dst_hbm, out_hbm, src_v, dst_v, stage,
               acc_sh, gsems, ssems):
    cid = lax.axis_index("c")
    sid = lax.axis_index("s")

    z16 = jnp.zeros((16,), jnp.float32)

    def zrow(r, _):
        for l in range(DH // 16):
            stage[0, r, pl.ds(l * 16, 16)] = z16
        return ()

    lax.fori_loop(0, K, zrow, ())
    for t in range(ROWS_PT // K):
        pltpu.sync_copy(stage.at[0],
                        acc_sh.at[pl.ds(sid * ROWS_PT + t * K, K)])
    plsc.subcore_barrier()

    pltpu.sync_copy(src_hbm.at[sid], src_v)
    pltpu.sync_copy(dst_hbm.at[sid], dst_v)
    zc = z_hbm.at[cid]

    def start_gather(j, b):
        pltpu.async_copy(zc.at[src_v.at[j]], stage.at[b], gsems.at[b])

    def wait_gather(j, b):
        pltpu.make_async_copy(zc.at[src_v.at[j]], stage.at[b],
                              gsems.at[b]).wait()

    def start_scat(j, b):
        pltpu.async_copy(stage.at[b], acc_sh.at[dst_v.at[j]], ssems.at[b],
                         add=True)

    def wait_scat(j, b):
        pltpu.make_async_copy(stage.at[b], acc_sh.at[dst_v.at[j]],
                              ssems.at[b]).wait()

    for b in range(NBUF):
        start_gather(b, b)

    def body(i, _):
        for b in range(NBUF):
            j = i * NBUF + b
            wait_gather(j, b)
            start_scat(j, b)
            wait_scat(j, b)
            start_gather(j + NBUF, b)
        return ()

    lax.fori_loop(0, NCHUNK // NBUF - 1, body, ())
    for b in range(NBUF):
        j = NCHUNK - NBUF + b
        wait_gather(j, b)
        start_scat(j, b)
        wait_scat(j, b)

    plsc.subcore_barrier()
    for t in range(ROWS_PT // K):
        sl = pl.ds(sid * ROWS_PT + t * K, K)
        pltpu.sync_copy(acc_sh.at[sl], out_hbm.at[cid].at[sl])


_spmm_kernel = pl.kernel(
    _spmm_body,
    out_type=jax.ShapeDtypeStruct((NC, NPAD, DH), jnp.float32),
    mesh=_mesh,
    scratch_types=[
        pltpu.VMEM((NCHUNK, K), jnp.int32),
        pltpu.VMEM((NCHUNK, K), jnp.int32),
        pltpu.VMEM((NBUF, K, DH), jnp.float32),
        pltpu.VMEM_SHARED((NPAD, DH), jnp.float32),
        pltpu.SemaphoreType.DMA((NBUF,)),
        pltpu.SemaphoreType.DMA((NBUF,)),
    ],
    compiler_params=pltpu.CompilerParams(use_tc_tiling_on_sc=False),
)


RB = 1024


def _norms(dp_ref):
    dsrc = dp_ref[0, 0, :] + dp_ref[1, 0, :]
    ddst = dp_ref[0, 1, :] + dp_ref[1, 1, :]
    ns = lax.rsqrt(jnp.maximum(dsrc, 1.0))
    nd = lax.rsqrt(jnp.maximum(ddst, 1.0))
    return ns, nd


def _tc1_body(x_ref, w_ref, dp_ref, o_ref):
    ns, _ = _norms(dp_ref)
    o_ref[0] = jnp.dot(x_ref[...], w_ref[0],
                       preferred_element_type=jnp.float32) * ns[:, None]


def _tc2_body(s_ref, dp_ref, b1_ref, w_ref, o_ref):
    ns, nd = _norms(dp_ref)
    s = jnp.concatenate([s_ref[0], s_ref[1]], axis=1)
    h = s * nd[:, None] + b1_ref[0][None, :]
    h = jnp.maximum(h, 0.0)
    o_ref[0] = jnp.dot(h, w_ref[0],
                       preferred_element_type=jnp.float32) * ns[:, None]


def _tc3_body(s_ref, dp_ref, b2_ref, o_ref):
    _, nd = _norms(dp_ref)
    s = jnp.concatenate([s_ref[0], s_ref[1]], axis=1)
    o_ref[...] = s * nd[:, None] + b2_ref[0][None, :]


_x_spec = pl.BlockSpec((RB, D), lambda i, c: (i, 0))
_wh_spec = pl.BlockSpec((1, D, DH), lambda i, c: (c, 0, 0))
_dp2_spec = pl.BlockSpec((NC, 2, RB), lambda i, c: (0, 0, i))
_b2d_spec = pl.BlockSpec((1, D), lambda i, c: (0, 0))
_sin_spec = pl.BlockSpec((NC, RB, DH), lambda i, c: (0, i, 0))
_oh_spec = pl.BlockSpec((1, RB, DH), lambda i, c: (c, i, 0))
_half_struct = jax.ShapeDtypeStruct((NC, NPAD, DH), jnp.float32)

_tc1 = pl.pallas_call(_tc1_body, grid=(NPAD // RB, NC),
                      in_specs=[_x_spec, _wh_spec, _dp2_spec],
                      out_specs=_oh_spec, out_shape=_half_struct)
_tc2 = pl.pallas_call(_tc2_body, grid=(NPAD // RB, NC),
                      in_specs=[_sin_spec, _dp2_spec, _b2d_spec, _wh_spec],
                      out_specs=_oh_spec, out_shape=_half_struct)
_tc3 = pl.pallas_call(
    _tc3_body, grid=(NPAD // RB,),
    in_specs=[pl.BlockSpec((NC, RB, DH), lambda i: (0, i, 0)),
              pl.BlockSpec((NC, 2, RB), lambda i: (0, 0, i)),
              pl.BlockSpec((1, D), lambda i: (0, 0))],
    out_specs=pl.BlockSpec((RB, D), lambda i: (i, 0)),
    out_shape=jax.ShapeDtypeStruct((NPAD, D), jnp.float32))


def kernel(x, edge_index, W1, b1, W2, b2):
    src = edge_index[0].astype(jnp.int32)
    dst = edge_index[1].astype(jnp.int32)
    npad_e = EPAD - N_EDGES
    pad_idx = N_NODES + (jnp.arange(npad_e, dtype=jnp.int32) % 16)
    src_p = jnp.concatenate([src, pad_idx]).reshape(NS, NCHUNK, K)
    dst_p = jnp.concatenate([dst, pad_idx]).reshape(NS, NCHUNK, K)

    x_p = jnp.pad(x, ((0, NPAD - N_NODES), (0, 0)))
    b1_2d = b1.reshape(1, D)
    b2_2d = b2.reshape(1, D)
    w1_s = jnp.moveaxis(W1.reshape(D, NC, DH), 1, 0)
    w2_s = jnp.moveaxis(W2.reshape(D, NC, DH), 1, 0)

    dp = _deg_kernel(src_p, dst_p)
    z1 = _tc1(x_p, w1_s, dp)
    s1 = _spmm_kernel(z1, src_p, dst_p)
    z2 = _tc2(s1, dp, b1_2d, w2_s)
    s2 = _spmm_kernel(z2, src_p, dst_p)
    out = _tc3(s2, dp, b2_2d)
    return out[:N_NODES]
